# baseline (speedup 1.0000x reference)
import functools

import jax
import jax.numpy as jnp
from jax import lax
from jax.experimental import pallas as pl
from jax.experimental.pallas import tpu as pltpu
from jax.experimental.pallas import tpu_sc as plsc

N = 10000
E = 320000
D = 128
H = 32

ROW_BLK = 1000


def _dense_body(x_ref, h_ref, c_ref, wi_ref, wf_ref, wc_ref, wo_ref,
                ti_ref, tf_ref, tc_ref, to_ref,
                tbi_ref, tbf_ref, tbc_ref, tbo_ref,
                bi_ref, bf_ref, bc_ref, bo_ref,
                wlin_ref, blin_ref, wsrc_ref, bsrc_ref, wdst_ref, bdst_ref,
                hnew_ref, cnew_ref, a_ref, b_ref):
    x = x_ref[...]
    h = h_ref[...]
    c = c_ref[...]
    wcat = jnp.concatenate(
        [wi_ref[...], wf_ref[...], wc_ref[...], wo_ref[...]], axis=1)
    tcat = jnp.concatenate(
        [ti_ref[...], tf_ref[...], tc_ref[...], to_ref[...]], axis=1)
    bias = jnp.concatenate(
        [tbi_ref[...] + bi_ref[...], tbf_ref[...] + bf_ref[...],
         tbc_ref[...] + bc_ref[...], tbo_ref[...] + bo_ref[...]], axis=1)
    g = (jnp.dot(x, wcat, preferred_element_type=jnp.float32)
         + jnp.dot(h, tcat, preferred_element_type=jnp.float32)
         + bias)
    i_g = jax.nn.sigmoid(g[:, 0:H])
    f_g = jax.nn.sigmoid(g[:, H:2 * H])
    t_g = jnp.tanh(g[:, 2 * H:3 * H])
    o_g = jax.nn.sigmoid(g[:, 3 * H:4 * H])
    c_new = f_g * c + i_g * t_g
    h_new = o_g * jnp.tanh(c_new)
    z = (jnp.dot(jax.nn.relu(h_new), wlin_ref[...],
                 preferred_element_type=jnp.float32) + blin_ref[...])
    wsd = jnp.concatenate([wsrc_ref[...], wdst_ref[...]], axis=1)
    bsd = jnp.concatenate([bsrc_ref[...], bdst_ref[...]], axis=1)
    ab = jnp.dot(z, wsd, preferred_element_type=jnp.float32) + bsd
    hnew_ref[...] = h_new
    cnew_ref[...] = c_new
    a_ref[...] = ab[:, 0:H]
    b_ref[...] = ab[:, H:2 * H]


def _dense_call(x, h0, c0, *weights):
    grid = N // ROW_BLK
    row_spec = lambda cols: pl.BlockSpec((ROW_BLK, cols), lambda i: (i, 0))
    full = lambda arr: pl.BlockSpec(arr.shape, lambda i: (0,) * arr.ndim)
    return pl.pallas_call(
        _dense_body,
        grid=(grid,),
        in_specs=[row_spec(D), row_spec(H), row_spec(H)]
                 + [full(w) for w in weights],
        out_specs=[row_spec(H), row_spec(H), row_spec(H), row_spec(H)],
        out_shape=[jax.ShapeDtypeStruct((N, H), jnp.float32)] * 4,
    )(x, h0, c0, *weights)


try:
    _INFO = plsc.get_sparse_core_info()
    _NC, _NS = _INFO.num_cores, _INFO.num_subcores
except Exception:
    _NC, _NS = 2, 16
_NW = _NC * _NS
CHUNK = 80
HP = H // 2
_PER_W = E // _NW
_NCHUNK = _PER_W // CHUNK
_NGRP = CHUNK // 16


def _edge_body(a_hbm, b_hbm, src_hbm, dst_hbm, neg_hbm, wb_hbm,
               pos_hbm, neg_out_hbm,
               w_v, src_v, dst_v, neg_v,
               a0, bd0, bn0, a1, bd1, bn1, a2, bd2, bn2,
               a3, bd3, bn3, a4, bd4, bn4,
               pos_all, negres_all, sem0, sem1, sem2, sem3, sem4):
    wid = lax.axis_index("s") * _NC + lax.axis_index("c")
    base = wid * _PER_W
    pltpu.sync_copy(wb_hbm, w_v)
    pltpu.sync_copy(src_hbm.at[pl.ds(base, _PER_W)], src_v)
    pltpu.sync_copy(dst_hbm.at[pl.ds(base, _PER_W)], dst_v)
    pltpu.sync_copy(neg_hbm.at[pl.ds(base, _PER_W)], neg_v)
    bufs = ((a0, bd0, bn0, sem0), (a1, bd1, bn1, sem1),
            (a2, bd2, bn2, sem2), (a3, bd3, bn3, sem3),
            (a4, bd4, bn4, sem4))

    def fire(k, which):
        a_r, bd_r, bn_r, sem = bufs[which]
        s = pl.ds(k * CHUNK, CHUNK)
        pltpu.async_copy(a_hbm.at[src_v.at[s]], a_r, sem)
        pltpu.async_copy(b_hbm.at[dst_v.at[s]], bd_r, sem)
        pltpu.async_copy(b_hbm.at[neg_v.at[s]], bn_r, sem)

    def wait(k, which):
        a_r, bd_r, bn_r, sem = bufs[which]
        s = pl.ds(k * CHUNK, CHUNK)
        pltpu.make_async_copy(a_hbm.at[src_v.at[s]], a_r, sem).wait()
        pltpu.make_async_copy(b_hbm.at[dst_v.at[s]], bd_r, sem).wait()
        pltpu.make_async_copy(b_hbm.at[neg_v.at[s]], bn_r, sem).wait()

    def compute(k, which):
        a_r, bd_r, bn_r, _ = bufs[which]

        def do_group(g):
            iota16 = lax.iota(jnp.int32, 16)
            rows = g * 16 + iota16

            def do_jblock(t, accs):
                acc_p, acc_n = accs
                for u in range(8):
                    j = t * 8 + u
                    col = jnp.bitwise_and(iota16 + j, H - 1)
                    av = plsc.load_gather(a_r, [rows, col])
                    bd = plsc.load_gather(bd_r, [rows, col])
                    bn = plsc.load_gather(bn_r, [rows, col])
                    wj = plsc.load_gather(w_v, [col])
                    acc_p = acc_p + jnp.maximum(av + bd, 0.0) * wj
                    acc_n = acc_n + jnp.maximum(av + bn, 0.0) * wj
                return acc_p, acc_n

            acc_p, acc_n = lax.fori_loop(
                0, 4, do_jblock,
                (jnp.zeros((16,), jnp.float32), jnp.zeros((16,), jnp.float32)))
            brow = plsc.load_gather(w_v, [H + iota16])
            out_s = pl.ds(k * CHUNK + g * 16, 16)
            pos_all[out_s] = 1.0 / (1.0 + jnp.exp(-(acc_p + brow)))
            negres_all[out_s] = 1.0 / (1.0 + jnp.exp(-(acc_n + brow)))

        for g in range(_NGRP):
            do_group(g)

    for b in range(4):
        fire(b, b)

    def do_penta(m, _):
        for b in range(5):
            k = 5 * m + b
            wait(k, b)
            kf = k + 4

            @pl.when(kf < _NCHUNK)
            def _():
                fire(kf, (b + 4) % 5)

            compute(k, b)
        return 0

    lax.fori_loop(0, _NCHUNK // 5, do_penta, 0)
    pltpu.sync_copy(pos_all, pos_hbm.at[pl.ds(base, _PER_W)])
    pltpu.sync_copy(negres_all, neg_out_hbm.at[pl.ds(base, _PER_W)])


def _edge_call(a_tbl, b_tbl, src, dst, neg, wpack):
    mesh = plsc.VectorSubcoreMesh(core_axis_name="c", subcore_axis_name="s")
    kfn = pl.kernel(
        _edge_body,
        out_type=[jax.ShapeDtypeStruct((E,), jnp.float32),
                  jax.ShapeDtypeStruct((E,), jnp.float32)],
        mesh=mesh,
        compiler_params=pltpu.CompilerParams(needs_layout_passes=False,
                                             use_tc_tiling_on_sc=False),
        scratch_types=[
            pltpu.VMEM((48,), jnp.float32),
            pltpu.VMEM((_PER_W,), jnp.int32),
            pltpu.VMEM((_PER_W,), jnp.int32),
            pltpu.VMEM((_PER_W,), jnp.int32),
        ] + [pltpu.VMEM((CHUNK, H), jnp.float32)] * 15
          + [
            pltpu.VMEM((_PER_W,), jnp.float32),
            pltpu.VMEM((_PER_W,), jnp.float32),
        ] + [pltpu.SemaphoreType.DMA] * 5,
    )
    return kfn(a_tbl, b_tbl, src, dst, neg, wpack)


def kernel(node_feat, src, dst, neg, edge_weight, h0, c0,
           W_i, W_f, W_c, W_o, T_i, T_f, T_c, T_o,
           tb_i, tb_f, tb_c, tb_o, b_i, b_f, b_c, b_o,
           W_lin, b_lin, W_src, b_src, W_dst, b_dst, W_out, b_out):
    del edge_weight
    h_new, c_new, a_tbl, b_tbl = _dense_call(
        node_feat, h0, c0,
        W_i, W_f, W_c, W_o, T_i, T_f, T_c, T_o,
        tb_i[None, :], tb_f[None, :], tb_c[None, :], tb_o[None, :],
        b_i, b_f, b_c, b_o,
        W_lin, b_lin[None, :], W_src, b_src[None, :], W_dst, b_dst[None, :])
    wb = jnp.concatenate([W_out.reshape(H), jnp.tile(b_out, 16)])
    pos_out, neg_out = _edge_call(
        a_tbl, b_tbl, src.astype(jnp.int32), dst.astype(jnp.int32),
        neg.astype(jnp.int32), wb)
    return (pos_out, neg_out, h_new, c_new)


# speedup vs baseline: 1.5320x; 1.5320x over previous
import functools

import jax
import jax.numpy as jnp
from jax import lax
from jax.experimental import pallas as pl
from jax.experimental.pallas import tpu as pltpu
from jax.experimental.pallas import tpu_sc as plsc

N = 10000
E = 320000
D = 128
H = 32

ROW_BLK = 1000


def _dense_body(x_ref, h_ref, c_ref, wi_ref, wf_ref, wc_ref, wo_ref,
                ti_ref, tf_ref, tc_ref, to_ref,
                tbi_ref, tbf_ref, tbc_ref, tbo_ref,
                bi_ref, bf_ref, bc_ref, bo_ref,
                wlin_ref, blin_ref, wsrc_ref, bsrc_ref, wdst_ref, bdst_ref,
                hnew_ref, cnew_ref, a_ref, b_ref):
    x = x_ref[...]
    h = h_ref[...]
    c = c_ref[...]
    wcat = jnp.concatenate(
        [wi_ref[...], wf_ref[...], wc_ref[...], wo_ref[...]], axis=1)
    tcat = jnp.concatenate(
        [ti_ref[...], tf_ref[...], tc_ref[...], to_ref[...]], axis=1)
    bias = jnp.concatenate(
        [tbi_ref[...] + bi_ref[...], tbf_ref[...] + bf_ref[...],
         tbc_ref[...] + bc_ref[...], tbo_ref[...] + bo_ref[...]], axis=1)
    g = (jnp.dot(x, wcat, preferred_element_type=jnp.float32)
         + jnp.dot(h, tcat, preferred_element_type=jnp.float32)
         + bias)
    i_g = jax.nn.sigmoid(g[:, 0:H])
    f_g = jax.nn.sigmoid(g[:, H:2 * H])
    t_g = jnp.tanh(g[:, 2 * H:3 * H])
    o_g = jax.nn.sigmoid(g[:, 3 * H:4 * H])
    c_new = f_g * c + i_g * t_g
    h_new = o_g * jnp.tanh(c_new)
    z = (jnp.dot(jax.nn.relu(h_new), wlin_ref[...],
                 preferred_element_type=jnp.float32) + blin_ref[...])
    wsd = jnp.concatenate([wsrc_ref[...], wdst_ref[...]], axis=1)
    bsd = jnp.concatenate([bsrc_ref[...], bdst_ref[...]], axis=1)
    ab = jnp.dot(z, wsd, preferred_element_type=jnp.float32) + bsd
    hnew_ref[...] = h_new
    cnew_ref[...] = c_new
    a_ref[...] = ab[:, 0:H]
    b_ref[...] = ab[:, H:2 * H]


def _dense_call(x, h0, c0, *weights):
    grid = N // ROW_BLK
    row_spec = lambda cols: pl.BlockSpec((ROW_BLK, cols), lambda i: (i, 0))
    full = lambda arr: pl.BlockSpec(arr.shape, lambda i: (0,) * arr.ndim)
    return pl.pallas_call(
        _dense_body,
        grid=(grid,),
        in_specs=[row_spec(D), row_spec(H), row_spec(H)]
                 + [full(w) for w in weights],
        out_specs=[row_spec(H), row_spec(H), row_spec(H), row_spec(H)],
        out_shape=[jax.ShapeDtypeStruct((N, H), jnp.float32)] * 4,
    )(x, h0, c0, *weights)


try:
    _INFO = plsc.get_sparse_core_info()
    _NC, _NS = _INFO.num_cores, _INFO.num_subcores
except Exception:
    _NC, _NS = 2, 16
_NW = _NC * _NS
CHUNK = 80
HP = H // 2
_PER_W = E // _NW
_NCHUNK = _PER_W // CHUNK
_NGRP = CHUNK // 16


def _edge_body(a_hbm, b_hbm, src_hbm, dst_hbm, neg_hbm, wb_hbm,
               pos_hbm, neg_out_hbm,
               w_v, src_v, dst_v, neg_v,
               a0, bd0, bn0, a1, bd1, bn1, a2, bd2, bn2,
               a3, bd3, bn3, a4, bd4, bn4,
               pos_all, negres_all, sem0, sem1, sem2, sem3, sem4):
    wid = lax.axis_index("s") * _NC + lax.axis_index("c")
    base = wid * _PER_W
    pltpu.sync_copy(wb_hbm, w_v)
    pltpu.sync_copy(src_hbm.at[pl.ds(base, _PER_W)], src_v)
    pltpu.sync_copy(dst_hbm.at[pl.ds(base, _PER_W)], dst_v)
    pltpu.sync_copy(neg_hbm.at[pl.ds(base, _PER_W)], neg_v)
    bufs = ((a0, bd0, bn0, sem0), (a1, bd1, bn1, sem1),
            (a2, bd2, bn2, sem2), (a3, bd3, bn3, sem3),
            (a4, bd4, bn4, sem4))

    def fire(k, which):
        a_r, bd_r, bn_r, sem = bufs[which]
        s = pl.ds(k * CHUNK, CHUNK)
        pltpu.async_copy(a_hbm.at[src_v.at[s]], a_r, sem)
        pltpu.async_copy(b_hbm.at[dst_v.at[s]], bd_r, sem)
        pltpu.async_copy(b_hbm.at[neg_v.at[s]], bn_r, sem)

    def wait(k, which):
        a_r, bd_r, bn_r, sem = bufs[which]
        s = pl.ds(k * CHUNK, CHUNK)
        pltpu.make_async_copy(a_hbm.at[src_v.at[s]], a_r, sem).wait()
        pltpu.make_async_copy(b_hbm.at[dst_v.at[s]], bd_r, sem).wait()
        pltpu.make_async_copy(b_hbm.at[neg_v.at[s]], bn_r, sem).wait()

    def compute(k, which):
        a_r, bd_r, bn_r, _ = bufs[which]

        def do_group(g, _):
            iota16 = lax.iota(jnp.int32, 16)
            rows = g * 16 + iota16

            def do_jblock(t, accs):
                acc_p, acc_n = accs
                for u in range(8):
                    j = t * 8 + u
                    col = jnp.bitwise_and(iota16 + j, H - 1)
                    av = plsc.load_gather(a_r, [rows, col])
                    bd = plsc.load_gather(bd_r, [rows, col])
                    bn = plsc.load_gather(bn_r, [rows, col])
                    wj = plsc.load_gather(w_v, [col])
                    acc_p = acc_p + jnp.maximum(av + bd, 0.0) * wj
                    acc_n = acc_n + jnp.maximum(av + bn, 0.0) * wj
                return acc_p, acc_n

            acc_p, acc_n = lax.fori_loop(
                0, 4, do_jblock,
                (jnp.zeros((16,), jnp.float32), jnp.zeros((16,), jnp.float32)))
            brow = plsc.load_gather(w_v, [H + iota16])
            out_s = pl.ds(k * CHUNK + g * 16, 16)
            pos_all[out_s] = 1.0 / (1.0 + jnp.exp(-(acc_p + brow)))
            negres_all[out_s] = 1.0 / (1.0 + jnp.exp(-(acc_n + brow)))
            return 0

        lax.fori_loop(0, _NGRP, do_group, 0)

    for b in range(4):
        fire(b, b)

    def do_penta(m, _):
        for b in range(5):
            k = 5 * m + b
            wait(k, b)
            kf = k + 4

            @pl.when(kf < _NCHUNK)
            def _():
                fire(kf, (b + 4) % 5)

            compute(k, b)
        return 0

    lax.fori_loop(0, _NCHUNK // 5, do_penta, 0)
    pltpu.sync_copy(pos_all, pos_hbm.at[pl.ds(base, _PER_W)])
    pltpu.sync_copy(negres_all, neg_out_hbm.at[pl.ds(base, _PER_W)])


def _edge_call(a_tbl, b_tbl, src, dst, neg, wpack):
    mesh = plsc.VectorSubcoreMesh(core_axis_name="c", subcore_axis_name="s")
    kfn = pl.kernel(
        _edge_body,
        out_type=[jax.ShapeDtypeStruct((E,), jnp.float32),
                  jax.ShapeDtypeStruct((E,), jnp.float32)],
        mesh=mesh,
        compiler_params=pltpu.CompilerParams(needs_layout_passes=False,
                                             use_tc_tiling_on_sc=False),
        scratch_types=[
            pltpu.VMEM((48,), jnp.float32),
            pltpu.VMEM((_PER_W,), jnp.int32),
            pltpu.VMEM((_PER_W,), jnp.int32),
            pltpu.VMEM((_PER_W,), jnp.int32),
        ] + [pltpu.VMEM((CHUNK, H), jnp.float32)] * 15
          + [
            pltpu.VMEM((_PER_W,), jnp.float32),
            pltpu.VMEM((_PER_W,), jnp.float32),
        ] + [pltpu.SemaphoreType.DMA] * 5,
    )
    return kfn(a_tbl, b_tbl, src, dst, neg, wpack)


def kernel(node_feat, src, dst, neg, edge_weight, h0, c0,
           W_i, W_f, W_c, W_o, T_i, T_f, T_c, T_o,
           tb_i, tb_f, tb_c, tb_o, b_i, b_f, b_c, b_o,
           W_lin, b_lin, W_src, b_src, W_dst, b_dst, W_out, b_out):
    del edge_weight
    h_new, c_new, a_tbl, b_tbl = _dense_call(
        node_feat, h0, c0,
        W_i, W_f, W_c, W_o, T_i, T_f, T_c, T_o,
        tb_i[None, :], tb_f[None, :], tb_c[None, :], tb_o[None, :],
        b_i, b_f, b_c, b_o,
        W_lin, b_lin[None, :], W_src, b_src[None, :], W_dst, b_dst[None, :])
    wb = jnp.concatenate([W_out.reshape(H), jnp.tile(b_out, 16)])
    pos_out, neg_out = _edge_call(
        a_tbl, b_tbl, src.astype(jnp.int32), dst.astype(jnp.int32),
        neg.astype(jnp.int32), wb)
    return (pos_out, neg_out, h_new, c_new)
